# baseline (speedup 1.0000x reference)
import functools

import jax
import jax.numpy as jnp
from jax import lax
from jax.experimental import pallas as pl
from jax.experimental.pallas import tpu as pltpu
from jax.experimental.pallas import tpu_sc as plsc

Q, K, D = 3136, 8192, 1536
B, H, W = 4, 28, 28
NN = 9

NCHUNK = 2
QC = Q // NCHUNK
NC, NS, L = 2, 16, 16
NWORK = NC * NS
RPW = QC // NWORK
SROW = 56

KT = 1024


def _dist_body(q_ref, m_ref, out_ref):
    q = q_ref[...]
    m = m_ref[...]
    q2 = jnp.sum(q * q, axis=1, keepdims=True)
    m2 = jnp.sum(m * m, axis=1)[None, :]
    dot = lax.dot_general(q, m, (((1,), (1,)), ((), ())),
                          preferred_element_type=jnp.float32)
    out_ref[...] = jnp.maximum(q2 + m2 - 2.0 * dot, 0.0)


def _distances(qf, mb, c):
    return pl.pallas_call(
        _dist_body,
        grid=(K // KT,),
        in_specs=[
            pl.BlockSpec((QC, D), lambda j, c=c: (c, 0)),
            pl.BlockSpec((KT, D), lambda j: (j, 0)),
        ],
        out_specs=pl.BlockSpec((QC, KT), lambda j: (0, j)),
        out_shape=jax.ShapeDtypeStruct((QC, K), jnp.float32),
    )(qf, mb)


def _sqrt_sc(x):
    xi = lax.bitcast_convert_type(x, jnp.int32)
    y = lax.bitcast_convert_type(
        jnp.int32(0x1FBD1DF5) + (xi >> 1), jnp.float32)
    for _ in range(3):
        y = 0.5 * (y + x / y)
    return y


def _score_body(d2_hbm, scores_hbm, wmax_hbm, row_a, row_b, cand_v, sc_v,
                mx_v, ip_v, sem0, sem1):
    wid = lax.axis_index("s") * NC + lax.axis_index("c")
    base = wid * RPW
    lane = jnp.arange(L, dtype=jnp.int32)
    inf_v = jnp.full((L,), jnp.inf, dtype=jnp.float32)
    bufs = (row_a, row_b)
    sems = (sem0, sem1)

    def fetch(row, b):
        pltpu.async_copy(d2_hbm.at[base + row], bufs[b], sems[b])

    def wait(b):
        pltpu.make_async_copy(d2_hbm.at[base], bufs[b], sems[b]).wait()

    def lane_min(r_ref):
        def p1(i, m):
            return jnp.minimum(m, r_ref[pl.ds(i * (4 * L), L)])
        m = plsc.parallel_loop(0, K // (4 * L), carry=inf_v, unroll=16)(p1)
        ms = jnp.sort(m)
        t = jnp.max(jnp.where(lane == 9, ms, -jnp.inf))
        return jnp.full((L,), t, dtype=jnp.float32)

    def scatter_pass(r_ref, t_v):
        def p2(i, ptr):
            v = r_ref[pl.ds(i * L, L)]
            msk = v <= t_v
            plsc.store_scatter(cand_v, [ptr], v, mask=msk)
            return ptr + jnp.where(msk, jnp.int32(L), jnp.int32(0))
        return plsc.parallel_loop(0, K // L, carry=lane, unroll=16)(p2)

    def process(r_ref, r, wm, t_v):
        del t_v
        ptr = scatter_pass(r_ref, lane_min(r_ref))

        maxc = jnp.max((ptr - lane) >> 4)
        def p3(j, best):
            v = cand_v[pl.ds(j * L, L)]
            vm = jnp.where(ptr > lane + j * L, v, jnp.inf)
            vs = jnp.sort(vm)
            low = jnp.minimum(best, lax.rev(vs, (0,)))
            return jnp.sort(low)
        best = lax.fori_loop(0, maxc, p3, inf_v)

        d = _sqrt_sc(best)
        d_m = jnp.where(lane < NN, d, -jnp.inf)
        dmax = jnp.max(d_m)
        s = jnp.where(lane < NN, jnp.exp(d - dmax), 0.0)
        ssum_v = jnp.full((L,), jnp.sum(s), dtype=jnp.float32)
        score_v = jnp.where(lane == 0, (1.0 - s / ssum_v) * d, -jnp.inf)
        plsc.store_scatter(sc_v, [jnp.full((L,), r, jnp.int32)], score_v,
                           mask=lane == 0)
        t_next = jnp.full((L,), jnp.max(best), dtype=jnp.float32)
        return jnp.maximum(wm, score_v), t_next

    fetch(0, 0)

    def row_pair(g, carry):
        wm, t_v = carry
        r0 = 2 * g
        fetch(r0 + 1, 1)
        wait(0)
        wm, t_v = process(bufs[0], r0, wm, t_v)
        fetch(r0 + 2, 0)
        wait(1)
        return process(bufs[1], r0 + 1, wm, t_v)

    wm, t_v = lax.fori_loop(
        0, RPW // 2, row_pair,
        (jnp.full((L,), -jnp.inf, dtype=jnp.float32),
         jnp.full((L,), -jnp.inf, dtype=jnp.float32)))
    wait(0)
    wm, _ = process(bufs[0], RPW - 1, wm, t_v)
    mx_v[...] = wm
    pltpu.sync_copy(mx_v, wmax_hbm.at[wid])
    pltpu.sync_copy(sc_v, scores_hbm.at[wid])


_score_call = functools.partial(
    pl.kernel,
    out_type=(
        jax.ShapeDtypeStruct((NWORK, SROW), jnp.float32),
        jax.ShapeDtypeStruct((NWORK, L), jnp.float32),
    ),
    mesh=plsc.VectorSubcoreMesh(core_axis_name="c", subcore_axis_name="s",
                                num_cores=NC, num_subcores=NS),
    compiler_params=pltpu.CompilerParams(needs_layout_passes=False),
    scratch_types=(
        pltpu.VMEM((K,), jnp.float32),
        pltpu.VMEM((K,), jnp.float32),
        pltpu.VMEM((K + 2 * L,), jnp.float32),
        pltpu.VMEM((SROW,), jnp.float32),
        pltpu.VMEM((L,), jnp.float32),
        pltpu.VMEM((L,), jnp.int32),
        pltpu.SemaphoreType.DMA,
        pltpu.SemaphoreType.DMA,
    ),
)(_score_body)


def kernel(patch_features, memory_bank):
    score_rows = []
    image_maxes = []
    for c in range(NCHUNK):
        d2 = _distances(patch_features, memory_bank, c)
        scores2d, wmax = _score_call(d2)
        score_rows.append(scores2d[:, :RPW].reshape(QC))
        image_maxes.append(jnp.max(wmax.reshape(QC // (H * W), -1), axis=1))
    score_map = jnp.concatenate(score_rows).reshape(B, H, W)
    image_scores = jnp.concatenate(image_maxes)
    return (image_scores, score_map)


# speedup vs baseline: 1.0028x; 1.0028x over previous
import functools

import jax
import jax.numpy as jnp
from jax import lax
from jax.experimental import pallas as pl
from jax.experimental.pallas import tpu as pltpu
from jax.experimental.pallas import tpu_sc as plsc

Q, K, D = 3136, 8192, 1536
B, H, W = 4, 28, 28
NN = 9

NCHUNK = 2
QC = Q // NCHUNK
NC, NS, L = 2, 16, 16
NWORK = NC * NS
RPW = QC // NWORK
SROW = 56

KT = 1024


def _dist_body(q_ref, m_ref, out_ref):
    q = q_ref[...]
    m = m_ref[...]
    q2 = jnp.sum(q * q, axis=1, keepdims=True)
    m2 = jnp.sum(m * m, axis=1)[None, :]
    dot = lax.dot_general(q, m, (((1,), (1,)), ((), ())),
                          preferred_element_type=jnp.float32)
    out_ref[...] = jnp.maximum(q2 + m2 - 2.0 * dot, 0.0)


def _distances(qf, mb, c):
    return pl.pallas_call(
        _dist_body,
        grid=(K // KT,),
        in_specs=[
            pl.BlockSpec((QC, D), lambda j, c=c: (c, 0)),
            pl.BlockSpec((KT, D), lambda j: (j, 0)),
        ],
        out_specs=pl.BlockSpec((QC, KT), lambda j: (0, j)),
        out_shape=jax.ShapeDtypeStruct((QC, K), jnp.float32),
    )(qf, mb)


def _sqrt_sc(x):
    xi = lax.bitcast_convert_type(x, jnp.int32)
    y = lax.bitcast_convert_type(
        jnp.int32(0x1FBD1DF5) + (xi >> 1), jnp.float32)
    for _ in range(3):
        y = 0.5 * (y + x / y)
    return y


def _score_body(d2_hbm, scores_hbm, wmax_hbm, row_a, row_b, cand_v, sc_v,
                mx_v, ip_v, sem0, sem1):
    wid = lax.axis_index("s") * NC + lax.axis_index("c")
    base = wid * RPW
    lane = jnp.arange(L, dtype=jnp.int32)
    inf_v = jnp.full((L,), jnp.inf, dtype=jnp.float32)
    bufs = (row_a, row_b)
    sems = (sem0, sem1)

    def fetch(row, b):
        pltpu.async_copy(d2_hbm.at[base + row], bufs[b], sems[b])

    def wait(b):
        pltpu.make_async_copy(d2_hbm.at[base], bufs[b], sems[b]).wait()

    def lane_min(r_ref):
        def p1(i, m):
            return jnp.minimum(m, r_ref[pl.ds(i * (4 * L), L)])
        m = plsc.parallel_loop(0, K // (4 * L), carry=inf_v, unroll=8)(p1)
        ms = jnp.sort(m)
        t = jnp.max(jnp.where(lane == 9, ms, -jnp.inf))
        return jnp.full((L,), t, dtype=jnp.float32)

    def scatter_pass(r_ref, t_v):
        def p2(i, ptr):
            v = r_ref[pl.ds(i * L, L)]
            msk = v <= t_v
            plsc.store_scatter(cand_v, [ptr], v, mask=msk)
            return ptr + jnp.where(msk, jnp.int32(L), jnp.int32(0))
        return plsc.parallel_loop(0, K // L, carry=lane, unroll=16)(p2)

    def process(r_ref, r, wm, t_v):
        del t_v
        ptr = scatter_pass(r_ref, lane_min(r_ref))

        maxc = jnp.max((ptr - lane) >> 4)
        def p3(j, best):
            v = cand_v[pl.ds(j * L, L)]
            vm = jnp.where(ptr > lane + j * L, v, jnp.inf)
            vs = jnp.sort(vm)
            low = jnp.minimum(best, lax.rev(vs, (0,)))
            return jnp.sort(low)
        best = lax.fori_loop(0, maxc, p3, inf_v)

        d = _sqrt_sc(best)
        d_m = jnp.where(lane < NN, d, -jnp.inf)
        dmax = jnp.max(d_m)
        s = jnp.where(lane < NN, jnp.exp(d - dmax), 0.0)
        ssum_v = jnp.full((L,), jnp.sum(s), dtype=jnp.float32)
        score_v = jnp.where(lane == 0, (1.0 - s / ssum_v) * d, -jnp.inf)
        plsc.store_scatter(sc_v, [jnp.full((L,), r, jnp.int32)], score_v,
                           mask=lane == 0)
        t_next = jnp.full((L,), jnp.max(best), dtype=jnp.float32)
        return jnp.maximum(wm, score_v), t_next

    fetch(0, 0)

    def row_pair(g, carry):
        wm, t_v = carry
        r0 = 2 * g
        fetch(r0 + 1, 1)
        wait(0)
        wm, t_v = process(bufs[0], r0, wm, t_v)
        fetch(r0 + 2, 0)
        wait(1)
        return process(bufs[1], r0 + 1, wm, t_v)

    wm, t_v = lax.fori_loop(
        0, RPW // 2, row_pair,
        (jnp.full((L,), -jnp.inf, dtype=jnp.float32),
         jnp.full((L,), -jnp.inf, dtype=jnp.float32)))
    wait(0)
    wm, _ = process(bufs[0], RPW - 1, wm, t_v)
    mx_v[...] = wm
    pltpu.sync_copy(mx_v, wmax_hbm.at[wid])
    pltpu.sync_copy(sc_v, scores_hbm.at[wid])


_score_call = functools.partial(
    pl.kernel,
    out_type=(
        jax.ShapeDtypeStruct((NWORK, SROW), jnp.float32),
        jax.ShapeDtypeStruct((NWORK, L), jnp.float32),
    ),
    mesh=plsc.VectorSubcoreMesh(core_axis_name="c", subcore_axis_name="s",
                                num_cores=NC, num_subcores=NS),
    compiler_params=pltpu.CompilerParams(needs_layout_passes=False),
    scratch_types=(
        pltpu.VMEM((K,), jnp.float32),
        pltpu.VMEM((K,), jnp.float32),
        pltpu.VMEM((K + 2 * L,), jnp.float32),
        pltpu.VMEM((SROW,), jnp.float32),
        pltpu.VMEM((L,), jnp.float32),
        pltpu.VMEM((L,), jnp.int32),
        pltpu.SemaphoreType.DMA,
        pltpu.SemaphoreType.DMA,
    ),
)(_score_body)


def kernel(patch_features, memory_bank):
    score_rows = []
    image_maxes = []
    for c in range(NCHUNK):
        d2 = _distances(patch_features, memory_bank, c)
        scores2d, wmax = _score_call(d2)
        score_rows.append(scores2d[:, :RPW].reshape(QC))
        image_maxes.append(jnp.max(wmax.reshape(QC // (H * W), -1), axis=1))
    score_map = jnp.concatenate(score_rows).reshape(B, H, W)
    image_scores = jnp.concatenate(image_maxes)
    return (image_scores, score_map)


# speedup vs baseline: 1.0030x; 1.0002x over previous
import functools

import jax
import jax.numpy as jnp
from jax import lax
from jax.experimental import pallas as pl
from jax.experimental.pallas import tpu as pltpu
from jax.experimental.pallas import tpu_sc as plsc

Q, K, D = 3136, 8192, 1536
B, H, W = 4, 28, 28
NN = 9

NCHUNK = 2
QC = Q // NCHUNK
NC, NS, L = 2, 16, 16
NWORK = NC * NS
RPW = QC // NWORK
SROW = 56

KT = 1024


def _dist_body(q_ref, m_ref, out_ref):
    q = q_ref[...]
    m = m_ref[...]
    q2 = jnp.sum(q * q, axis=1, keepdims=True)
    m2 = jnp.sum(m * m, axis=1)[None, :]
    dot = lax.dot_general(q, m, (((1,), (1,)), ((), ())),
                          preferred_element_type=jnp.float32)
    out_ref[...] = jnp.maximum(q2 + m2 - 2.0 * dot, 0.0)


def _distances(qf, mb, c):
    return pl.pallas_call(
        _dist_body,
        grid=(K // KT,),
        in_specs=[
            pl.BlockSpec((QC, D), lambda j, c=c: (c, 0)),
            pl.BlockSpec((KT, D), lambda j: (j, 0)),
        ],
        out_specs=pl.BlockSpec((QC, KT), lambda j: (0, j)),
        out_shape=jax.ShapeDtypeStruct((QC, K), jnp.float32),
    )(qf, mb)


def _sqrt_sc(x):
    xi = lax.bitcast_convert_type(x, jnp.int32)
    y = lax.bitcast_convert_type(
        jnp.int32(0x1FBD1DF5) + (xi >> 1), jnp.float32)
    for _ in range(3):
        y = 0.5 * (y + x / y)
    return y


def _score_body(d2_hbm, scores_hbm, wmax_hbm, row_a, row_b, cand_v, sc_v,
                mx_v, sem0, sem1):
    wid = lax.axis_index("s") * NC + lax.axis_index("c")
    base = wid * RPW
    lane = jnp.arange(L, dtype=jnp.int32)
    inf_v = jnp.full((L,), jnp.inf, dtype=jnp.float32)
    bufs = (row_a, row_b)
    sems = (sem0, sem1)

    def fetch(row, b):
        pltpu.async_copy(d2_hbm.at[base + row], bufs[b], sems[b])

    def wait(b):
        pltpu.make_async_copy(d2_hbm.at[base], bufs[b], sems[b]).wait()

    def lane_min(r_ref):
        def p1(i, m):
            return jnp.minimum(m, r_ref[pl.ds(i * (4 * L), L)])
        m = plsc.parallel_loop(0, K // (4 * L), carry=inf_v, unroll=8)(p1)
        ms = jnp.sort(m)
        t = jnp.max(jnp.where(lane == 9, ms, -jnp.inf))
        return jnp.full((L,), t, dtype=jnp.float32)

    def scatter_pass(r_ref, t_v):
        def p2(i, ptr):
            v = r_ref[pl.ds(i * L, L)]
            msk = v <= t_v
            plsc.store_scatter(cand_v, [ptr], v, mask=msk)
            return ptr + jnp.where(msk, jnp.int32(L), jnp.int32(0))
        return plsc.parallel_loop(0, K // L, carry=lane, unroll=16)(p2)

    def process(r_ref, r, wm):
        ptr = scatter_pass(r_ref, lane_min(r_ref))

        maxc = jnp.max((ptr - lane) >> 4)
        def p3(j, best):
            v = cand_v[pl.ds(j * L, L)]
            vm = jnp.where(ptr > lane + j * L, v, jnp.inf)
            vs = jnp.sort(vm)
            low = jnp.minimum(best, lax.rev(vs, (0,)))
            return jnp.sort(low)
        best = lax.fori_loop(0, maxc, p3, inf_v)

        d = _sqrt_sc(best)
        d_m = jnp.where(lane < NN, d, -jnp.inf)
        dmax = jnp.max(d_m)
        s = jnp.where(lane < NN, jnp.exp(d - dmax), 0.0)
        ssum_v = jnp.full((L,), jnp.sum(s), dtype=jnp.float32)
        score_v = jnp.where(lane == 0, (1.0 - s / ssum_v) * d, -jnp.inf)
        plsc.store_scatter(sc_v, [jnp.full((L,), r, jnp.int32)], score_v,
                           mask=lane == 0)
        return jnp.maximum(wm, score_v)

    fetch(0, 0)

    def row_pair(g, wm):
        r0 = 2 * g
        fetch(r0 + 1, 1)
        wait(0)
        wm = process(bufs[0], r0, wm)
        fetch(r0 + 2, 0)
        wait(1)
        return process(bufs[1], r0 + 1, wm)

    wm = lax.fori_loop(0, RPW // 2, row_pair,
                       jnp.full((L,), -jnp.inf, dtype=jnp.float32))
    wait(0)
    wm = process(bufs[0], RPW - 1, wm)
    mx_v[...] = wm
    pltpu.sync_copy(mx_v, wmax_hbm.at[wid])
    pltpu.sync_copy(sc_v, scores_hbm.at[wid])


_score_call = functools.partial(
    pl.kernel,
    out_type=(
        jax.ShapeDtypeStruct((NWORK, SROW), jnp.float32),
        jax.ShapeDtypeStruct((NWORK, L), jnp.float32),
    ),
    mesh=plsc.VectorSubcoreMesh(core_axis_name="c", subcore_axis_name="s",
                                num_cores=NC, num_subcores=NS),
    compiler_params=pltpu.CompilerParams(needs_layout_passes=False),
    scratch_types=(
        pltpu.VMEM((K,), jnp.float32),
        pltpu.VMEM((K,), jnp.float32),
        pltpu.VMEM((K + 2 * L,), jnp.float32),
        pltpu.VMEM((SROW,), jnp.float32),
        pltpu.VMEM((L,), jnp.float32),
        pltpu.SemaphoreType.DMA,
        pltpu.SemaphoreType.DMA,
    ),
)(_score_body)


def kernel(patch_features, memory_bank):
    score_rows = []
    image_maxes = []
    for c in range(NCHUNK):
        d2 = _distances(patch_features, memory_bank, c)
        scores2d, wmax = _score_call(d2)
        score_rows.append(scores2d[:, :RPW].reshape(QC))
        image_maxes.append(jnp.max(wmax.reshape(QC // (H * W), -1), axis=1))
    score_map = jnp.concatenate(score_rows).reshape(B, H, W)
    image_scores = jnp.concatenate(image_maxes)
    return (image_scores, score_map)
